# baseline (speedup 1.0000x reference)
import jax
import jax.numpy as jnp
from jax.experimental import pallas as pl
from jax.experimental.pallas import tpu as pltpu

B, N = 1024, 512
ENT_DIM, KEY_DIM, UT_DIM, FUNC_DIM, IN_DIM = 256, 32, 259, 256, 1024
BB = 32


def _body(emb_ref, autm_ref, mask_ref, ee_ref,
          wk_ref, bk_ref, wf_ref, bf_ref, w1_ref, b1_ref, w2_ref, b2_ref,
          logits_ref, idx_ref):
    f32 = jnp.float32
    x = jnp.maximum(jnp.dot(emb_ref[...], w1_ref[...],
                            preferred_element_type=f32) + b1_ref[...], 0.0)
    fe = jnp.maximum(jnp.dot(autm_ref[...], wf_ref[...],
                             preferred_element_type=f32) + bf_ref[...], 0.0)
    q = jnp.maximum(jnp.dot(x + fe, w2_ref[...],
                            preferred_element_type=f32) + b2_ref[...], 0.0)

    ee2 = ee_ref[...].reshape(BB * N, ENT_DIM)
    key = jnp.dot(ee2, wk_ref[...], preferred_element_type=f32) + bk_ref[...]
    key3 = key.reshape(BB, N, KEY_DIM)

    lg = jnp.mean(key3 * q[:, None, :], axis=2)
    lg = lg - (1.0 - mask_ref[...]) * 1000000000.0
    logits_ref[...] = lg

    m = jnp.max(lg, axis=1, keepdims=True)
    ii = jax.lax.broadcasted_iota(jnp.int32, (BB, N), 1)
    cand = jnp.where(lg == m, ii, N)
    idx_ref[...] = jnp.min(cand, axis=1, keepdims=True)


def kernel(embedding, available_unit_type_mask, available_units_mask,
           entity_embedding, W_key, b_key, W_func, b_func,
           W_fc1, b_fc1, W_fc2, b_fc2):
    f32 = jnp.float32
    bk = b_key.reshape(1, KEY_DIM)
    bf = b_func.reshape(1, FUNC_DIM)
    b1 = b_fc1.reshape(1, FUNC_DIM)
    b2 = b_fc2.reshape(1, KEY_DIM)

    rep = lambda shape: pl.BlockSpec(shape, lambda i: (0,) * len(shape))
    logits, idx = pl.pallas_call(
        _body,
        grid=(B // BB,),
        in_specs=[
            pl.BlockSpec((BB, IN_DIM), lambda i: (i, 0)),
            pl.BlockSpec((BB, UT_DIM), lambda i: (i, 0)),
            pl.BlockSpec((BB, N), lambda i: (i, 0)),
            pl.BlockSpec((BB, N, ENT_DIM), lambda i: (i, 0, 0)),
            rep((ENT_DIM, KEY_DIM)),
            rep((1, KEY_DIM)),
            rep((UT_DIM, FUNC_DIM)),
            rep((1, FUNC_DIM)),
            rep((IN_DIM, FUNC_DIM)),
            rep((1, FUNC_DIM)),
            rep((FUNC_DIM, KEY_DIM)),
            rep((1, KEY_DIM)),
        ],
        out_specs=[
            pl.BlockSpec((BB, N), lambda i: (i, 0)),
            pl.BlockSpec((BB, 1), lambda i: (i, 0)),
        ],
        out_shape=[
            jax.ShapeDtypeStruct((B, N), f32),
            jax.ShapeDtypeStruct((B, 1), jnp.int32),
        ],
        compiler_params=pltpu.CompilerParams(
            dimension_semantics=("arbitrary",),
        ),
    )(embedding, available_unit_type_mask, available_units_mask,
      entity_embedding, W_key, bk, W_func, bf, W_fc1, b1, W_fc2, b2)
    return (logits, idx)


# speedup vs baseline: 2.1531x; 2.1531x over previous
import jax
import jax.numpy as jnp
from jax.experimental import pallas as pl
from jax.experimental.pallas import tpu as pltpu

B, N = 1024, 512
ENT_DIM, KEY_DIM, UT_DIM, FUNC_DIM, IN_DIM = 256, 32, 259, 256, 1024
BB = 32


def _body(emb_ref, autm_ref, mask_ref, ee_ref,
          wk_ref, bk_ref, wf_ref, bf_ref, w1_ref, b1_ref, w2_ref, b2_ref,
          logits_ref, idx_ref):
    f32 = jnp.float32
    x = jnp.maximum(jnp.dot(emb_ref[...], w1_ref[...],
                            preferred_element_type=f32) + b1_ref[...], 0.0)
    fe = jnp.maximum(jnp.dot(autm_ref[...], wf_ref[...],
                             preferred_element_type=f32) + bf_ref[...], 0.0)
    q = jnp.maximum(jnp.dot(x + fe, w2_ref[...],
                            preferred_element_type=f32) + b2_ref[...], 0.0)

    v = jax.lax.dot_general(q, wk_ref[...], (((1,), (1,)), ((), ())),
                            preferred_element_type=f32)
    c = jnp.dot(q, bk_ref[...].reshape(KEY_DIM, 1),
                preferred_element_type=f32)

    ee2 = ee_ref[...].reshape(BB * N, ENT_DIM)
    lgT = jax.lax.dot_general(v, ee2, (((1,), (1,)), ((), ())),
                              preferred_element_type=f32)
    lg3 = lgT.reshape(BB, BB, N)
    e0 = jax.lax.broadcasted_iota(jnp.int32, (BB, BB, 1), 0)
    e1 = jax.lax.broadcasted_iota(jnp.int32, (BB, BB, 1), 1)
    eye = (e0 == e1).astype(f32)
    lg = jnp.sum(lg3 * eye, axis=1)
    lg = (lg + c) * (1.0 / KEY_DIM)
    lg = lg - (1.0 - mask_ref[...]) * 1000000000.0
    logits_ref[...] = lg

    m = jnp.max(lg, axis=1, keepdims=True)
    ii = jax.lax.broadcasted_iota(jnp.int32, (BB, N), 1)
    cand = jnp.where(lg == m, ii, N)
    idx_ref[...] = jnp.min(cand, axis=1, keepdims=True)


def kernel(embedding, available_unit_type_mask, available_units_mask,
           entity_embedding, W_key, b_key, W_func, b_func,
           W_fc1, b_fc1, W_fc2, b_fc2):
    f32 = jnp.float32
    bk = b_key.reshape(1, KEY_DIM)
    bf = b_func.reshape(1, FUNC_DIM)
    b1 = b_fc1.reshape(1, FUNC_DIM)
    b2 = b_fc2.reshape(1, KEY_DIM)

    rep = lambda shape: pl.BlockSpec(shape, lambda i: (0,) * len(shape))
    logits, idx = pl.pallas_call(
        _body,
        grid=(B // BB,),
        in_specs=[
            pl.BlockSpec((BB, IN_DIM), lambda i: (i, 0)),
            pl.BlockSpec((BB, UT_DIM), lambda i: (i, 0)),
            pl.BlockSpec((BB, N), lambda i: (i, 0)),
            pl.BlockSpec((BB, N, ENT_DIM), lambda i: (i, 0, 0)),
            rep((ENT_DIM, KEY_DIM)),
            rep((1, KEY_DIM)),
            rep((UT_DIM, FUNC_DIM)),
            rep((1, FUNC_DIM)),
            rep((IN_DIM, FUNC_DIM)),
            rep((1, FUNC_DIM)),
            rep((FUNC_DIM, KEY_DIM)),
            rep((1, KEY_DIM)),
        ],
        out_specs=[
            pl.BlockSpec((BB, N), lambda i: (i, 0)),
            pl.BlockSpec((BB, 1), lambda i: (i, 0)),
        ],
        out_shape=[
            jax.ShapeDtypeStruct((B, N), f32),
            jax.ShapeDtypeStruct((B, 1), jnp.int32),
        ],
        compiler_params=pltpu.CompilerParams(
            dimension_semantics=("arbitrary",),
        ),
    )(embedding, available_unit_type_mask, available_units_mask,
      entity_embedding, W_key, bk, W_func, bf, W_fc1, b1, W_fc2, b2)
    return (logits, idx)


# speedup vs baseline: 2.1623x; 1.0043x over previous
import jax
import jax.numpy as jnp
from jax.experimental import pallas as pl
from jax.experimental.pallas import tpu as pltpu

B, N = 1024, 512
ENT_DIM, KEY_DIM, UT_DIM, FUNC_DIM, IN_DIM = 256, 32, 259, 256, 1024
BB = 32


def _body(emb_ref, autm_ref, mask_ref, ee_ref,
          wk_ref, bk_ref, wf_ref, bf_ref, w1_ref, b1_ref, w2_ref, b2_ref,
          logits_ref, idx_ref):
    f32 = jnp.float32
    x = jnp.maximum(jnp.dot(emb_ref[...], w1_ref[...],
                            preferred_element_type=f32) + b1_ref[...], 0.0)
    fe = jnp.maximum(jnp.dot(autm_ref[...], wf_ref[...],
                             preferred_element_type=f32) + bf_ref[...], 0.0)
    q = jnp.maximum(jnp.dot(x + fe, w2_ref[...],
                            preferred_element_type=f32) + b2_ref[...], 0.0)

    ee2 = ee_ref[...].reshape(BB * N, ENT_DIM)
    wkT = jnp.transpose(wk_ref[...])
    keyT = jax.lax.dot_general(wkT, ee2, (((1,), (1,)), ((), ())),
                               preferred_element_type=f32)
    keyT = keyT + jnp.transpose(bk_ref[...])
    key3 = keyT.reshape(KEY_DIM, BB, N)
    qT = jnp.transpose(q)
    lg = jnp.mean(key3 * qT[:, :, None], axis=0)
    lg = lg - (1.0 - mask_ref[...]) * 1000000000.0
    logits_ref[...] = lg

    m = jnp.max(lg, axis=1, keepdims=True)
    ii = jax.lax.broadcasted_iota(jnp.int32, (BB, N), 1)
    cand = jnp.where(lg == m, ii, N)
    idx_ref[...] = jnp.min(cand, axis=1, keepdims=True)


def kernel(embedding, available_unit_type_mask, available_units_mask,
           entity_embedding, W_key, b_key, W_func, b_func,
           W_fc1, b_fc1, W_fc2, b_fc2):
    f32 = jnp.float32
    bk = b_key.reshape(1, KEY_DIM)
    bf = b_func.reshape(1, FUNC_DIM)
    b1 = b_fc1.reshape(1, FUNC_DIM)
    b2 = b_fc2.reshape(1, KEY_DIM)

    rep = lambda shape: pl.BlockSpec(shape, lambda i: (0,) * len(shape))
    logits, idx = pl.pallas_call(
        _body,
        grid=(B // BB,),
        in_specs=[
            pl.BlockSpec((BB, IN_DIM), lambda i: (i, 0)),
            pl.BlockSpec((BB, UT_DIM), lambda i: (i, 0)),
            pl.BlockSpec((BB, N), lambda i: (i, 0)),
            pl.BlockSpec((BB, N, ENT_DIM), lambda i: (i, 0, 0)),
            rep((ENT_DIM, KEY_DIM)),
            rep((1, KEY_DIM)),
            rep((UT_DIM, FUNC_DIM)),
            rep((1, FUNC_DIM)),
            rep((IN_DIM, FUNC_DIM)),
            rep((1, FUNC_DIM)),
            rep((FUNC_DIM, KEY_DIM)),
            rep((1, KEY_DIM)),
        ],
        out_specs=[
            pl.BlockSpec((BB, N), lambda i: (i, 0)),
            pl.BlockSpec((BB, 1), lambda i: (i, 0)),
        ],
        out_shape=[
            jax.ShapeDtypeStruct((B, N), f32),
            jax.ShapeDtypeStruct((B, 1), jnp.int32),
        ],
        compiler_params=pltpu.CompilerParams(
            dimension_semantics=("arbitrary",),
        ),
    )(embedding, available_unit_type_mask, available_units_mask,
      entity_embedding, W_key, bk, W_func, bf, W_fc1, b1, W_fc2, b2)
    return (logits, idx)
